# baseline (speedup 1.0000x reference)
import jax
import jax.numpy as jnp
from jax.experimental import pallas as pl
from jax.experimental.pallas import tpu as pltpu
from jax.experimental.pallas import tpu_sc as plsc

_B, _S, _J, _C, _TL = 16, 2048, 512, 10, 64
_TS = 1024
_NT_IN = _S // _TS
_NT = (_S + 1 + _TS - 1) // _TS
_W = 88
_FB = 256
_NW = 32
_HROWS = _S // 2


def _merge_window(base, rel0, sos_row, lab, nrows, wrows):
    if nrows == 65:
        blk = jnp.concatenate([sos_row, lab], axis=0)
    else:
        blk = jnp.concatenate([sos_row, lab, sos_row], axis=0)
    rows = jax.lax.broadcasted_iota(jnp.int32, (wrows, nrows), 0) + rel0
    cols = jax.lax.broadcasted_iota(jnp.int32, (wrows, nrows), 1)
    oh = (rows == cols).astype(jnp.float32)
    repl = jax.lax.dot_general(
        oh, blk, (((1,), (0,)), ((), ())),
        precision=jax.lax.Precision.HIGHEST,
        preferred_element_type=jnp.float32)
    rel = rel0 + jax.lax.broadcasted_iota(jnp.int32, (wrows, 1), 0)
    mask = (rel >= 0) & (rel < nrows)
    return jnp.where(mask, repl, base)


def _x_body(lens_ref, c_ref, x_ref, sos_ref, lab_ref, ox_ref, carry_ref):
    b = pl.program_id(0)
    t = pl.program_id(1)
    lb = lens_ref[b]
    xb = x_ref[0]

    @pl.when(t == 0)
    def _():
        carry_ref[...] = sos_ref[0]

    ox_ref[0, 0:1, :] = carry_ref[...]
    ox_ref[0, 1:_TS, :] = xb[:_TS - 1, :]

    a = lb + 1 - t * _TS
    overlap = (lb + 66 > t * _TS) & (lb + 1 < t * _TS + _TS)

    @pl.when(overlap)
    def _():
        w = pl.multiple_of(jnp.clip((a // 8) * 8, 0, _TS - _W), 8)
        bw = ox_ref[0, pl.ds(w, _W), :]
        ox_ref[0, pl.ds(w, _W), :] = _merge_window(
            bw, w - a, sos_ref[0], lab_ref[0], 65, _W)

    carry_ref[...] = xb[_TS - 1:_TS, :]


def _sc_copy_body(tgt_hbm, ot_hbm):
    wid = jax.lax.axis_index("c") * 16 + jax.lax.axis_index("s")
    for w in range(_NW):
        b, h = w // 2, w % 2

        @pl.when(wid == w)
        def _():
            pltpu.sync_copy(
                tgt_hbm.at[pl.ds(b, 1), pl.ds(h * _HROWS, _HROWS), :],
                ot_hbm.at[pl.ds(b, 1), pl.ds(h * _HROWS, _HROWS), :])


def _sc_copy(tgt):
    mesh = plsc.VectorSubcoreMesh(
        core_axis_name="c", subcore_axis_name="s",
        num_cores=2, num_subcores=16)
    return pl.kernel(
        _sc_copy_body,
        out_type=jax.ShapeDtypeStruct((_B, _S, _J), jnp.float32),
        mesh=mesh,
    )(tgt)


def _fix_body(lens_ref, c_ref, ot_in_ref, sos_ref, lab_ref, ot_ref):
    b = pl.program_id(0)
    t = pl.program_id(1)
    lb = lens_ref[b]
    kb = jnp.minimum(lb // _FB + t, _S // _FB - 1)
    ot_ref[0] = _merge_window(
        ot_in_ref[0], kb * _FB - lb, sos_ref[0], lab_ref[0], 66, _FB)


def kernel(x, tgt, lens, c, sos, labels):
    sos3 = sos[:, None, :]
    x_grid = pltpu.PrefetchScalarGridSpec(
        num_scalar_prefetch=2,
        grid=(_B, _NT),
        in_specs=[
            pl.BlockSpec((1, _TS, _J),
                         lambda b, t, lens_ref, c_ref:
                         (b, jnp.minimum(t, _NT_IN - 1), 0)),
            pl.BlockSpec((1, 1, _J), lambda b, t, lens_ref, c_ref: (b, 0, 0)),
            pl.BlockSpec((1, _TL, _J),
                         lambda b, t, lens_ref, c_ref: (c_ref[b], 0, 0)),
        ],
        out_specs=pl.BlockSpec((1, _TS, _J),
                               lambda b, t, lens_ref, c_ref: (b, t, 0)),
        scratch_shapes=[pltpu.VMEM((1, _J), jnp.float32)],
    )
    out_x = pl.pallas_call(
        _x_body,
        grid_spec=x_grid,
        out_shape=jax.ShapeDtypeStruct((_B, _S + 1, _J), jnp.float32),
    )(lens, c, x, sos3, labels)

    tgt_copied = _sc_copy(tgt)

    def _fb_index(b, t, lens_ref, c_ref):
        return (b, jnp.minimum(lens_ref[b] // _FB + t, _S // _FB - 1), 0)

    fix_grid = pltpu.PrefetchScalarGridSpec(
        num_scalar_prefetch=2,
        grid=(_B, 2),
        in_specs=[
            pl.BlockSpec((1, _FB, _J), _fb_index),
            pl.BlockSpec((1, 1, _J), lambda b, t, lens_ref, c_ref: (b, 0, 0)),
            pl.BlockSpec((1, _TL, _J),
                         lambda b, t, lens_ref, c_ref: (c_ref[b], 0, 0)),
        ],
        out_specs=pl.BlockSpec((1, _FB, _J), _fb_index),
    )
    out_tgt = pl.pallas_call(
        _fix_body,
        grid_spec=fix_grid,
        out_shape=jax.ShapeDtypeStruct((_B, _S, _J), jnp.float32),
        input_output_aliases={2: 0},
    )(lens, c, tgt_copied, sos3, labels)

    return (out_x, out_tgt, labels)


# speedup vs baseline: 9.9839x; 9.9839x over previous
import jax
import jax.numpy as jnp
from jax.experimental import pallas as pl
from jax.experimental.pallas import tpu as pltpu
from jax.experimental.pallas import tpu_sc as plsc

_B, _S, _J, _C, _TL = 16, 2048, 512, 10, 64
_TS = 1024
_NT_IN = _S // _TS
_NT = (_S + 1 + _TS - 1) // _TS
_W = 88
_FB = 256
_NW = 32
_HROWS = _S // 2


def _merge_window(base, rel0, sos_row, lab, nrows, wrows):
    if nrows == 65:
        blk = jnp.concatenate([sos_row, lab], axis=0)
    else:
        blk = jnp.concatenate([sos_row, lab, sos_row], axis=0)
    rows = jax.lax.broadcasted_iota(jnp.int32, (wrows, nrows), 0) + rel0
    cols = jax.lax.broadcasted_iota(jnp.int32, (wrows, nrows), 1)
    oh = (rows == cols).astype(jnp.float32)
    repl = jax.lax.dot_general(
        oh, blk, (((1,), (0,)), ((), ())),
        precision=jax.lax.Precision.HIGHEST,
        preferred_element_type=jnp.float32)
    rel = rel0 + jax.lax.broadcasted_iota(jnp.int32, (wrows, 1), 0)
    mask = (rel >= 0) & (rel < nrows)
    return jnp.where(mask, repl, base)


def _x_body(lens_ref, c_ref, x_ref, sos_ref, lab_ref, ox_ref, carry_ref):
    b = pl.program_id(0)
    t = pl.program_id(1)
    lb = lens_ref[b]
    xb = x_ref[0]

    @pl.when(t == 0)
    def _():
        carry_ref[...] = sos_ref[0]

    ox_ref[0, 0:1, :] = carry_ref[...]
    ox_ref[0, 1:_TS, :] = xb[:_TS - 1, :]

    a = lb + 1 - t * _TS
    overlap = (lb + 66 > t * _TS) & (lb + 1 < t * _TS + _TS)

    @pl.when(overlap)
    def _():
        w = pl.multiple_of(jnp.clip((a // 8) * 8, 0, _TS - _W), 8)
        bw = ox_ref[0, pl.ds(w, _W), :]
        ox_ref[0, pl.ds(w, _W), :] = _merge_window(
            bw, w - a, sos_ref[0], lab_ref[0], 65, _W)

    carry_ref[...] = xb[_TS - 1:_TS, :]


_CH = 64
_NCH = _HROWS // _CH


def _sc_copy_body(tgt_hbm, ot_hbm, buf, rsem, wsem):
    wid = jax.lax.axis_index("c") * 16 + jax.lax.axis_index("s")
    b = wid // 2
    h = wid % 2

    def _rd(i, k):
        row = pl.multiple_of(h * _HROWS + i * _CH, 8)
        return pltpu.make_async_copy(
            tgt_hbm.at[pl.ds(b, 1), pl.ds(row, _CH), :], buf.at[k],
            rsem.at[k])

    def _wr(i, k):
        row = pl.multiple_of(h * _HROWS + i * _CH, 8)
        return pltpu.make_async_copy(
            buf.at[k], ot_hbm.at[pl.ds(b, 1), pl.ds(row, _CH), :],
            wsem.at[k])

    _rd(0, 0).start()
    for i in range(_NCH):
        k = i % 2
        _rd(i, k).wait()
        _wr(i, k).start()
        if i + 1 < _NCH:
            if i >= 1:
                _wr(i - 1, 1 - k).wait()
            _rd(i + 1, 1 - k).start()
    _wr(_NCH - 1, (_NCH - 1) % 2).wait()


def _sc_copy(tgt):
    mesh = plsc.VectorSubcoreMesh(
        core_axis_name="c", subcore_axis_name="s",
        num_cores=2, num_subcores=16)
    return pl.kernel(
        _sc_copy_body,
        out_type=jax.ShapeDtypeStruct((_B, _S, _J), jnp.float32),
        mesh=mesh,
        scratch_types=[
            pltpu.VMEM((2, 1, _CH, _J), jnp.float32),
            pltpu.SemaphoreType.DMA((2,)),
            pltpu.SemaphoreType.DMA((2,)),
        ],
    )(tgt)


def _fix_body(lens_ref, c_ref, ot_in_ref, sos_ref, lab_ref, ot_ref):
    b = pl.program_id(0)
    t = pl.program_id(1)
    lb = lens_ref[b]
    kb = jnp.minimum(lb // _FB + t, _S // _FB - 1)
    ot_ref[0] = _merge_window(
        ot_in_ref[0], kb * _FB - lb, sos_ref[0], lab_ref[0], 66, _FB)


def kernel(x, tgt, lens, c, sos, labels):
    sos3 = sos[:, None, :]
    x_grid = pltpu.PrefetchScalarGridSpec(
        num_scalar_prefetch=2,
        grid=(_B, _NT),
        in_specs=[
            pl.BlockSpec((1, _TS, _J),
                         lambda b, t, lens_ref, c_ref:
                         (b, jnp.minimum(t, _NT_IN - 1), 0)),
            pl.BlockSpec((1, 1, _J), lambda b, t, lens_ref, c_ref: (b, 0, 0)),
            pl.BlockSpec((1, _TL, _J),
                         lambda b, t, lens_ref, c_ref: (c_ref[b], 0, 0)),
        ],
        out_specs=pl.BlockSpec((1, _TS, _J),
                               lambda b, t, lens_ref, c_ref: (b, t, 0)),
        scratch_shapes=[pltpu.VMEM((1, _J), jnp.float32)],
    )
    out_x = pl.pallas_call(
        _x_body,
        grid_spec=x_grid,
        out_shape=jax.ShapeDtypeStruct((_B, _S + 1, _J), jnp.float32),
    )(lens, c, x, sos3, labels)

    tgt_copied = _sc_copy(tgt)

    def _fb_index(b, t, lens_ref, c_ref):
        return (b, jnp.minimum(lens_ref[b] // _FB + t, _S // _FB - 1), 0)

    fix_grid = pltpu.PrefetchScalarGridSpec(
        num_scalar_prefetch=2,
        grid=(_B, 2),
        in_specs=[
            pl.BlockSpec((1, _FB, _J), _fb_index),
            pl.BlockSpec((1, 1, _J), lambda b, t, lens_ref, c_ref: (b, 0, 0)),
            pl.BlockSpec((1, _TL, _J),
                         lambda b, t, lens_ref, c_ref: (c_ref[b], 0, 0)),
        ],
        out_specs=pl.BlockSpec((1, _FB, _J), _fb_index),
    )
    out_tgt = pl.pallas_call(
        _fix_body,
        grid_spec=fix_grid,
        out_shape=jax.ShapeDtypeStruct((_B, _S, _J), jnp.float32),
        input_output_aliases={2: 0},
    )(lens, c, tgt_copied, sos3, labels)

    return (out_x, out_tgt, labels)


# speedup vs baseline: 9.9862x; 1.0002x over previous
import jax
import jax.numpy as jnp
from jax.experimental import pallas as pl
from jax.experimental.pallas import tpu as pltpu
from jax.experimental.pallas import tpu_sc as plsc

_B, _S, _J, _C, _TL = 16, 2048, 512, 10, 64
_TS = 1024
_NT_IN = _S // _TS
_NT = (_S + 1 + _TS - 1) // _TS
_W = 88
_FB = 256
_NW = 32
_HROWS = _S // 2


def _merge_window(base, rel0, sos_row, lab, nrows, wrows):
    if nrows == 65:
        blk = jnp.concatenate([sos_row, lab], axis=0)
    else:
        blk = jnp.concatenate([sos_row, lab, sos_row], axis=0)
    rows = jax.lax.broadcasted_iota(jnp.int32, (wrows, nrows), 0) + rel0
    cols = jax.lax.broadcasted_iota(jnp.int32, (wrows, nrows), 1)
    oh = (rows == cols).astype(jnp.float32)
    repl = jax.lax.dot_general(
        oh, blk, (((1,), (0,)), ((), ())),
        precision=jax.lax.Precision.HIGHEST,
        preferred_element_type=jnp.float32)
    rel = rel0 + jax.lax.broadcasted_iota(jnp.int32, (wrows, 1), 0)
    mask = (rel >= 0) & (rel < nrows)
    return jnp.where(mask, repl, base)


def _x_body(lens_ref, c_ref, x_ref, sos_ref, lab_ref, ox_ref, carry_ref):
    b = pl.program_id(0)
    t = pl.program_id(1)
    lb = lens_ref[b]
    xb = x_ref[0]

    @pl.when(t == 0)
    def _():
        carry_ref[...] = sos_ref[0]

    ox_ref[0, 0:1, :] = carry_ref[...]
    ox_ref[0, 1:_TS, :] = xb[:_TS - 1, :]

    a = lb + 1 - t * _TS
    overlap = (lb + 66 > t * _TS) & (lb + 1 < t * _TS + _TS)

    @pl.when(overlap)
    def _():
        w = pl.multiple_of(jnp.clip((a // 8) * 8, 0, _TS - _W), 8)
        bw = ox_ref[0, pl.ds(w, _W), :]
        ox_ref[0, pl.ds(w, _W), :] = _merge_window(
            bw, w - a, sos_ref[0], lab_ref[0], 65, _W)

    carry_ref[...] = xb[_TS - 1:_TS, :]


_CH = 64
_NCH = _HROWS // _CH


def _sc_copy_body(tgt_hbm, ot_hbm, buf, rsem, wsem):
    wid = jax.lax.axis_index("c") * 16 + jax.lax.axis_index("s")
    b = wid // 2
    h = wid % 2

    def _rd(i, k):
        row = pl.multiple_of(h * _HROWS + i * _CH, 8)
        return pltpu.make_async_copy(
            tgt_hbm.at[pl.ds(b, 1), pl.ds(row, _CH), :], buf.at[k],
            rsem.at[k])

    def _wr(i, k):
        row = pl.multiple_of(h * _HROWS + i * _CH, 8)
        return pltpu.make_async_copy(
            buf.at[k], ot_hbm.at[pl.ds(b, 1), pl.ds(row, _CH), :],
            wsem.at[k])

    _rd(0, 0).start()
    for i in range(_NCH):
        k = i % 2
        _rd(i, k).wait()
        _wr(i, k).start()
        if i + 1 < _NCH:
            if i >= 1:
                _wr(i - 1, 1 - k).wait()
            _rd(i + 1, 1 - k).start()
    _wr(_NCH - 1, (_NCH - 1) % 2).wait()


def _sc_copy(tgt):
    mesh = plsc.VectorSubcoreMesh(
        core_axis_name="c", subcore_axis_name="s",
        num_cores=2, num_subcores=16)
    return pl.kernel(
        _sc_copy_body,
        out_type=jax.ShapeDtypeStruct((_B, _S, _J), jnp.float32),
        mesh=mesh,
        scratch_types=[
            pltpu.VMEM((2, 1, _CH, _J), jnp.float32),
            pltpu.SemaphoreType.DMA((2,)),
            pltpu.SemaphoreType.DMA((2,)),
        ],
    )(tgt)


def _fix_body(lens_ref, c_ref, ot_in_ref, sos_ref, lab_ref, ot_ref):
    b = pl.program_id(0)
    t = pl.program_id(1)
    lb = lens_ref[b]
    kb = jnp.minimum(lb // _FB + t, _S // _FB - 1)
    ot_ref[0] = _merge_window(
        ot_in_ref[0], kb * _FB - lb, sos_ref[0], lab_ref[0], 66, _FB)


def kernel(x, tgt, lens, c, sos, labels):
    sos3 = sos[:, None, :]
    tgt_copied = _sc_copy(tgt)
    x_grid = pltpu.PrefetchScalarGridSpec(
        num_scalar_prefetch=2,
        grid=(_B, _NT),
        in_specs=[
            pl.BlockSpec((1, _TS, _J),
                         lambda b, t, lens_ref, c_ref:
                         (b, jnp.minimum(t, _NT_IN - 1), 0)),
            pl.BlockSpec((1, 1, _J), lambda b, t, lens_ref, c_ref: (b, 0, 0)),
            pl.BlockSpec((1, _TL, _J),
                         lambda b, t, lens_ref, c_ref: (c_ref[b], 0, 0)),
        ],
        out_specs=pl.BlockSpec((1, _TS, _J),
                               lambda b, t, lens_ref, c_ref: (b, t, 0)),
        scratch_shapes=[pltpu.VMEM((1, _J), jnp.float32)],
    )
    out_x = pl.pallas_call(
        _x_body,
        grid_spec=x_grid,
        out_shape=jax.ShapeDtypeStruct((_B, _S + 1, _J), jnp.float32),
    )(lens, c, x, sos3, labels)

    def _fb_index(b, t, lens_ref, c_ref):
        return (b, jnp.minimum(lens_ref[b] // _FB + t, _S // _FB - 1), 0)

    fix_grid = pltpu.PrefetchScalarGridSpec(
        num_scalar_prefetch=2,
        grid=(_B, 2),
        in_specs=[
            pl.BlockSpec((1, _FB, _J), _fb_index),
            pl.BlockSpec((1, 1, _J), lambda b, t, lens_ref, c_ref: (b, 0, 0)),
            pl.BlockSpec((1, _TL, _J),
                         lambda b, t, lens_ref, c_ref: (c_ref[b], 0, 0)),
        ],
        out_specs=pl.BlockSpec((1, _FB, _J), _fb_index),
    )
    out_tgt = pl.pallas_call(
        _fix_body,
        grid_spec=fix_grid,
        out_shape=jax.ShapeDtypeStruct((_B, _S, _J), jnp.float32),
        input_output_aliases={2: 0},
    )(lens, c, tgt_copied, sos3, labels)

    return (out_x, out_tgt, labels)


# speedup vs baseline: 14.1825x; 1.4202x over previous
import jax
import jax.numpy as jnp
from jax.experimental import pallas as pl
from jax.experimental.pallas import tpu as pltpu

_B, _S, _J, _C, _TL = 16, 2048, 512, 10, 64
_H = _S // 2
_W = 88


def _merge_window(base, rel0, sos_row, lab, nrows):
    if nrows == 65:
        blk = jnp.concatenate([sos_row, lab], axis=0)
    else:
        blk = jnp.concatenate([sos_row, lab, sos_row], axis=0)
    rows = jax.lax.broadcasted_iota(jnp.int32, (_W, nrows), 0) + rel0
    cols = jax.lax.broadcasted_iota(jnp.int32, (_W, nrows), 1)
    oh = (rows == cols).astype(jnp.float32)
    repl = jax.lax.dot_general(
        oh, blk, (((1,), (0,)), ((), ())),
        precision=jax.lax.Precision.HIGHEST,
        preferred_element_type=jnp.float32)
    rel = rel0 + jax.lax.broadcasted_iota(jnp.int32, (_W, 1), 0)
    mask = (rel >= 0) & (rel < nrows)
    return jnp.where(mask, repl, base)


def _body(lens_ref, c_ref, xa_ref, xb_ref, sos_ref, lab_ref, ta_ref, tb_ref,
          ox_ref, ot_ref):
    b = pl.program_id(0)
    lb = lens_ref[b]

    ox_ref[0, 0:1, :] = sos_ref[0]
    ox_ref[0, 1:_H + 1, :] = xa_ref[0, 0]
    ox_ref[0, _H + 1:_S + 1, :] = xb_ref[0, 0]
    wx = pl.multiple_of(jnp.clip(((lb + 1) // 8) * 8, 0, _S - _W), 8)
    bw = ox_ref[0, pl.ds(wx, _W), :]
    ox_ref[0, pl.ds(wx, _W), :] = _merge_window(
        bw, wx - (lb + 1), sos_ref[0], lab_ref[0], 65)

    ot_ref[0, 0:_H, :] = ta_ref[0, 0]
    ot_ref[0, _H:_S, :] = tb_ref[0, 0]
    wt = pl.multiple_of(jnp.clip((lb // 8) * 8, 0, _S - _W), 8)
    bt = ot_ref[0, pl.ds(wt, _W), :]
    ot_ref[0, pl.ds(wt, _W), :] = _merge_window(
        bt, wt - lb, sos_ref[0], lab_ref[0], 66)


def kernel(x, tgt, lens, c, sos, labels):
    sos3 = sos[:, None, :]
    x2 = x.reshape(_B, 2, _H, _J)
    t2 = tgt.reshape(_B, 2, _H, _J)
    grid_spec = pltpu.PrefetchScalarGridSpec(
        num_scalar_prefetch=2,
        grid=(_B,),
        in_specs=[
            pl.BlockSpec((1, 1, _H, _J), lambda b, l, c: (b, 0, 0, 0)),
            pl.BlockSpec((1, 1, _H, _J), lambda b, l, c: (b, 1, 0, 0)),
            pl.BlockSpec((1, 1, _J), lambda b, l, c: (b, 0, 0)),
            pl.BlockSpec((1, _TL, _J), lambda b, l, c: (c[b], 0, 0)),
            pl.BlockSpec((1, 1, _H, _J), lambda b, l, c: (b, 0, 0, 0)),
            pl.BlockSpec((1, 1, _H, _J), lambda b, l, c: (b, 1, 0, 0)),
        ],
        out_specs=[
            pl.BlockSpec((1, _S + 1, _J), lambda b, l, c: (b, 0, 0)),
            pl.BlockSpec((1, _S, _J), lambda b, l, c: (b, 0, 0)),
        ],
    )
    out_x, out_tgt = pl.pallas_call(
        _body,
        grid_spec=grid_spec,
        out_shape=[
            jax.ShapeDtypeStruct((_B, _S + 1, _J), jnp.float32),
            jax.ShapeDtypeStruct((_B, _S, _J), jnp.float32),
        ],
    )(lens, c, x2, x2, sos3, labels, t2, t2)
    return (out_x, out_tgt, labels)


# speedup vs baseline: 14.2121x; 1.0021x over previous
import jax
import jax.numpy as jnp
from jax.experimental import pallas as pl
from jax.experimental.pallas import tpu as pltpu

_B, _S, _J, _C, _TL = 16, 2048, 512, 10, 64
_H = _S // 2
_Q = _S // 4
_W = 88


def _merge_window(base, rel0, sos_row, lab, nrows):
    if nrows == 65:
        blk = jnp.concatenate([sos_row, lab], axis=0)
    else:
        blk = jnp.concatenate([sos_row, lab, sos_row], axis=0)
    rows = jax.lax.broadcasted_iota(jnp.int32, (_W, nrows), 0) + rel0
    cols = jax.lax.broadcasted_iota(jnp.int32, (_W, nrows), 1)
    oh = (rows == cols).astype(jnp.float32)
    repl = jax.lax.dot_general(
        oh, blk, (((1,), (0,)), ((), ())),
        precision=jax.lax.Precision.HIGHEST,
        preferred_element_type=jnp.float32)
    rel = rel0 + jax.lax.broadcasted_iota(jnp.int32, (_W, 1), 0)
    mask = (rel >= 0) & (rel < nrows)
    return jnp.where(mask, repl, base)


def _body(lens_ref, c_ref, xa_ref, xb_ref, xc_ref, xd_ref, sos_ref, lab_ref,
          ta_ref, tb_ref, tc_ref, td_ref, ox_ref, ot_ref):
    b = pl.program_id(0)
    lb = lens_ref[b]

    ox_ref[0, 0:1, :] = sos_ref[0]
    ox_ref[0, 1:_Q + 1, :] = xa_ref[0, 0]
    ox_ref[0, _Q + 1:2 * _Q + 1, :] = xb_ref[0, 0]
    ox_ref[0, 2 * _Q + 1:3 * _Q + 1, :] = xc_ref[0, 0]
    ox_ref[0, 3 * _Q + 1:_S + 1, :] = xd_ref[0, 0]
    wx = pl.multiple_of(jnp.clip(((lb + 1) // 8) * 8, 0, _S - _W), 8)
    bw = ox_ref[0, pl.ds(wx, _W), :]
    ox_ref[0, pl.ds(wx, _W), :] = _merge_window(
        bw, wx - (lb + 1), sos_ref[0], lab_ref[0], 65)

    ot_ref[0, 0:_Q, :] = ta_ref[0, 0]
    ot_ref[0, _Q:2 * _Q, :] = tb_ref[0, 0]
    ot_ref[0, 2 * _Q:3 * _Q, :] = tc_ref[0, 0]
    ot_ref[0, 3 * _Q:_S, :] = td_ref[0, 0]
    wt = pl.multiple_of(jnp.clip((lb // 8) * 8, 0, _S - _W), 8)
    bt = ot_ref[0, pl.ds(wt, _W), :]
    ot_ref[0, pl.ds(wt, _W), :] = _merge_window(
        bt, wt - lb, sos_ref[0], lab_ref[0], 66)


def kernel(x, tgt, lens, c, sos, labels):
    sos3 = sos[:, None, :]
    x2 = x.reshape(_B, 4, _Q, _J)
    t2 = tgt.reshape(_B, 4, _Q, _J)
    grid_spec = pltpu.PrefetchScalarGridSpec(
        num_scalar_prefetch=2,
        grid=(_B,),
        in_specs=(
            [pl.BlockSpec((1, 1, _Q, _J),
                          (lambda q: lambda b, l, c: (b, q, 0, 0))(q))
             for q in range(4)]
            + [pl.BlockSpec((1, 1, _J), lambda b, l, c: (b, 0, 0)),
               pl.BlockSpec((1, _TL, _J), lambda b, l, c: (c[b], 0, 0))]
            + [pl.BlockSpec((1, 1, _Q, _J),
                            (lambda q: lambda b, l, c: (b, q, 0, 0))(q))
               for q in range(4)]),
        out_specs=[
            pl.BlockSpec((1, _S + 1, _J), lambda b, l, c: (b, 0, 0)),
            pl.BlockSpec((1, _S, _J), lambda b, l, c: (b, 0, 0)),
        ],
    )
    out_x, out_tgt = pl.pallas_call(
        _body,
        grid_spec=grid_spec,
        out_shape=[
            jax.ShapeDtypeStruct((_B, _S + 1, _J), jnp.float32),
            jax.ShapeDtypeStruct((_B, _S, _J), jnp.float32),
        ],
    )(lens, c, x2, x2, x2, x2, sos3, labels, t2, t2, t2, t2)
    return (out_x, out_tgt, labels)
